# baseline (speedup 1.0000x reference)
import jax
import jax.numpy as jnp
from jax.experimental import pallas as pl

N_GRAPHS = 512
HEADS1 = 10
D_IN = 35
D_OUT = 128


def _head_body(g_ref, fcw_ref, fcb_ref, ow_ref, ob_ref, out_ref):
    g = g_ref[...]
    h = jnp.maximum(jnp.dot(g, fcw_ref[...], preferred_element_type=jnp.float32)
                    + fcb_ref[...][None, :], 0.0)
    out_ref[...] = jnp.dot(h, ow_ref[...], preferred_element_type=jnp.float32) + ob_ref[...][None, :]


def _gat_conv(x, src, dst, W, att_src, att_dst, bias, heads, out_dim):
    N = x.shape[0]
    h = (x @ W).reshape(N, heads, out_dim)
    a_src = (h * att_src[None, :, :]).sum(-1)
    a_dst = (h * att_dst[None, :, :]).sum(-1)
    alpha = a_src[src] + a_dst[dst]
    alpha = jax.nn.leaky_relu(alpha, negative_slope=0.2)
    amax = jax.ops.segment_max(alpha, dst, num_segments=N)
    alpha = jnp.exp(alpha - amax[dst])
    denom = jax.ops.segment_sum(alpha, dst, num_segments=N)
    alpha = alpha / (denom[dst] + 1e-16)
    msg = h[src] * alpha[:, :, None]
    out = jax.ops.segment_sum(msg, dst, num_segments=N)
    return out.reshape(N, heads * out_dim) + bias


def kernel(x, edge_index, batch, W1, att_src1, att_dst1, b1, W2, att_src2, att_dst2, b2, fc_W, fc_b, out_W, out_b):
    N = x.shape[0]
    loop = jnp.arange(N, dtype=edge_index.dtype)
    src = jnp.concatenate([edge_index[0], loop])
    dst = jnp.concatenate([edge_index[1], loop])
    h = _gat_conv(x, src, dst, W1, att_src1, att_dst1, b1, HEADS1, D_IN)
    h = jax.nn.elu(h)
    h = _gat_conv(h, src, dst, W2, att_src2, att_dst2, b2, 1, D_OUT)
    h = jax.nn.relu(h)
    g = jax.ops.segment_max(h, batch, num_segments=N_GRAPHS)
    out = pl.pallas_call(
        _head_body,
        out_shape=jax.ShapeDtypeStruct((N_GRAPHS, 1), jnp.float32),
    )(g, fc_W, fc_b, out_W, out_b)
    return out


# speedup vs baseline: 21.8514x; 21.8514x over previous
import functools

import jax
import jax.numpy as jnp
from jax import lax
from jax.experimental import pallas as pl
from jax.experimental.pallas import tpu as pltpu
from jax.experimental.pallas import tpu_sc as plsc

N = 100000
G = 512
E_RAW = 1600000
E = E_RAW + N
NTILES = 32
GPT = 3328
EPW = GPT * 16
E_PAD = EPW * NTILES
E_ALLOC = E_PAD + 256
SHIFT = 10
BUCKET_N = 1 << SHIFT
NBUCK = 98
NBINS = 128
DUMB = 127
PADV = 1 << 28
L = 16

HW1, WP1 = 352, 368
HW2, WP2 = 128, 144
CHC = 128

mesh = plsc.VectorSubcoreMesh(core_axis_name="c", subcore_axis_name="s")
_scp = pltpu.CompilerParams(use_tc_tiling_on_sc=False)


def _hist_body(dst_ref, out_ref):
    d = dst_ref[0]
    b = jnp.minimum(jax.lax.shift_right_logical(d, SHIFT), DUMB)
    rows = [jnp.sum((b == k).astype(jnp.int32), axis=0) for k in range(NBINS)]
    out_ref[0] = jnp.stack(rows)


def _offsets_body(cnt_ref, offs_ref, bstart_ref):
    cnt = cnt_ref[...]
    r = lax.broadcasted_iota(jnp.int32, (512, 512), 0)
    c = lax.broadcasted_iota(jnp.int32, (512, 512), 1)
    lt = (c < r).astype(jnp.float32)
    excl = jnp.dot(lt, cnt, preferred_element_type=jnp.float32)
    tot = jnp.sum(cnt, axis=0, keepdims=True)
    ku = lax.broadcasted_iota(jnp.int32, (NBINS, NBINS), 0)
    bu = lax.broadcasted_iota(jnp.int32, (NBINS, NBINS), 1)
    up = (ku < bu).astype(jnp.float32)
    bstart = jnp.dot(tot, up, preferred_element_type=jnp.float32)
    offs_ref[...] = (excl + bstart).astype(jnp.int32)
    bstart_ref[...] = jnp.broadcast_to(bstart, (8, NBINS)).astype(jnp.int32)


def _cmax_body(ms_ref, md_ref, c_ref):
    m1 = jnp.max(ms_ref[...], axis=0)
    m2 = jnp.max(md_ref[...], axis=0)
    s = m1 + m2
    c_ref[...] = jnp.maximum(s, 0.2 * s)


def _mm1_body(x_ref, w1_ref, as_ref, ad_ref, h1_ref, a1s_ref, a1d_ref,
              ms_ref, md_ref):
    h = jnp.dot(x_ref[...], w1_ref[...], preferred_element_type=jnp.float32)
    a_s = jnp.dot(h, as_ref[...], preferred_element_type=jnp.float32)
    a_d = jnp.dot(h, ad_ref[...], preferred_element_type=jnp.float32)
    h1_ref[...] = h
    a1s_ref[...] = a_s
    a1d_ref[...] = a_d
    ms_ref[0] = jnp.broadcast_to(jnp.max(a_s, axis=0)[None, :], (8, 16))
    md_ref[0] = jnp.broadcast_to(jnp.max(a_d, axis=0)[None, :], (8, 16))


def _norm1mm2_body(acc_ref, rep_ref, b1_ref, w2_ref, a2s_ref, a2d_ref,
                   h2_ref, s2_ref, d2_ref, ms_ref, md_ref):
    a = acc_ref[...]
    den = jnp.dot(a[:, 352:368], rep_ref[...],
                  preferred_element_type=jnp.float32)
    msg = a[:, :352] / (den + 1e-16)
    x = msg + b1_ref[0:1, :]
    hl1 = jnp.where(x > 0, x, jnp.exp(jnp.minimum(x, 0.0)) - 1.0)
    h2 = jnp.dot(hl1, w2_ref[...], preferred_element_type=jnp.float32)
    a_s = jnp.dot(h2, a2s_ref[...], preferred_element_type=jnp.float32)
    a_d = jnp.dot(h2, a2d_ref[...], preferred_element_type=jnp.float32)
    h2_ref[...] = h2
    s2_ref[...] = a_s
    d2_ref[...] = a_d
    ms_ref[0] = jnp.broadcast_to(jnp.max(a_s, axis=0)[None, :], (8, 16))
    md_ref[0] = jnp.broadcast_to(jnp.max(a_d, axis=0)[None, :], (8, 16))


def _norm2_body(acc_ref, rep_ref, b2_ref, out_ref):
    a = acc_ref[...]
    den = jnp.dot(a[:, 128:144], rep_ref[...],
                  preferred_element_type=jnp.float32)
    x = a[:, :128] / (den + 1e-16) + b2_ref[0:1, :]
    out_ref[...] = jnp.maximum(x, 0.0)


def _head_body(p_ref, fcw_ref, fcb_ref, ow_ref, ob_ref, out_ref):
    g = jnp.max(p_ref[...], axis=0)
    fc = jnp.maximum(
        jnp.dot(g, fcw_ref[...], preferred_element_type=jnp.float32)
        + fcb_ref[0:1, :], 0.0)
    out_ref[...] = (jnp.dot(fc, ow_ref[...], preferred_element_type=jnp.float32)
                    + ob_ref[0:1, :])


BLIST = list(range(NBUCK)) + [DUMB]


@functools.partial(
    pl.kernel,
    out_type=[jax.ShapeDtypeStruct((E_ALLOC,), jnp.int32),
              jax.ShapeDtypeStruct((E_ALLOC,), jnp.int32)],
    mesh=mesh,
    scratch_types=[
        pltpu.VMEM((4096,), jnp.int32),
        pltpu.VMEM((4096,), jnp.int32),
        pltpu.VMEM((32, 128), jnp.int32),
        pltpu.VMEM((NBINS, 16), jnp.int32),
        pltpu.SemaphoreType.DMA,
        pltpu.SemaphoreType.DMA,
    ],
    compiler_params=_scp,
)
def _permute(src_hbm, dst_hbm, offs_hbm, psrc_hbm, pdst_hbm,
             srcb, dstb, posb, slab, sem, sem2):
    c = lax.axis_index("c")
    s = lax.axis_index("s")
    w = c * 16 + s
    base = w * EPW
    pltpu.sync_copy(offs_hbm.at[w], slab)
    for ch in range(13):
        e0 = base + ch * 4096
        pltpu.sync_copy(src_hbm.at[pl.ds(e0, 4096)], srcb)
        pltpu.sync_copy(dst_hbm.at[pl.ds(e0, 4096)], dstb)

        def body(g, _):
            dv = dstb[pl.ds(g * L, L)]
            b = jnp.minimum(jax.lax.shift_right_logical(dv, SHIFT), DUMB)
            pos = jnp.zeros((L,), jnp.int32)
            for bb in BLIST:
                cv = slab[bb, :]
                m = b == bb
                pos = jnp.where(m, cv, pos)
                slab[bb, :] = cv + jnp.where(m, 1, 0)
            posb[g >> 3, pl.ds((g & 7) * L, L)] = pos
            return _

        lax.fori_loop(0, 256, body, 0)
        for j in range(32):
            cp1 = pltpu.async_copy(srcb.at[pl.ds(j * 128, 128)],
                                   psrc_hbm.at[posb.at[j]], sem)
            cp2 = pltpu.async_copy(dstb.at[pl.ds(j * 128, 128)],
                                   pdst_hbm.at[posb.at[j]], sem2)
            cp1.wait()
            cp2.wait()


def _make_accum(HW, WP, heads):
    nchunk = HW // L
    zrows = 16

    @functools.partial(
        pl.kernel,
        out_type=jax.ShapeDtypeStruct((NBUCK * BUCKET_N, WP), jnp.float32),
        mesh=mesh,
        scratch_types=[
            pltpu.VMEM((CHC,), jnp.int32),
            pltpu.VMEM((CHC,), jnp.int32),
            pltpu.VMEM((CHC,), jnp.int32),
            pltpu.VMEM((CHC, HW), jnp.float32),
            pltpu.VMEM((CHC, 16), jnp.float32),
            pltpu.VMEM((CHC, 16), jnp.float32),
            pltpu.VMEM((CHC, WP), jnp.float32),
            pltpu.VMEM((NBINS,), jnp.int32),
            pltpu.VMEM((16,), jnp.float32),
            pltpu.VMEM((zrows, WP), jnp.float32),
            pltpu.VMEM_SHARED((BUCKET_N + 16, WP), jnp.float32),
            pltpu.SemaphoreType.DMA,
            pltpu.SemaphoreType.DMA,
            pltpu.SemaphoreType.DMA,
            pltpu.SemaphoreType.DMA,
        ],
        compiler_params=_scp,
    )
    def accum(psrc_hbm, pdst_hbm, h_hbm, as_hbm, ad_hbm, bs_hbm, c_hbm,
              out_hbm, srcb, dstb, locb, hb, asb, adb, updb, bsb, cbuf,
              zb, acc, sem, sem2, sem3, sem4):
        c = lax.axis_index("c")
        s = lax.axis_index("s")
        srows = BUCKET_N // 16
        pltpu.sync_copy(bs_hbm, bsb)
        pltpu.sync_copy(c_hbm.at[0], cbuf)
        cb16 = cbuf[...]
        iota = lax.broadcasted_iota(jnp.int32, (L,), 0)
        for j in range(zrows):
            for k in range(WP // L):
                zb[j, pl.ds(k * L, L)] = jnp.zeros((L,), jnp.float32)
        def bucket(b, _):
            @pl.when(c == (b % 2))
            def _process():
                bsv = bsb[pl.ds(b, 16)]
                lo = bsv[0]
                hi = bsv[1]
                for j in range(srows // zrows):
                    pltpu.sync_copy(
                        zb, acc.at[pl.ds(s * srows + j * zrows, zrows)])
                plsc.subcore_barrier()
                g0 = lax.shift_right_logical(lo, 4)
                g1 = lax.shift_right_logical(hi + 15, 4)
                ct = lax.shift_right_logical(g1 - g0 + 7, 3)
                nct = lax.shift_right_logical(ct + 15, 4)
                my0 = s * nct
                myn = jnp.clip(ct - my0, 0, nct)

                def chunk(i, _):
                    e0 = g0 * 16 + (my0 + i) * CHC
                    pltpu.sync_copy(psrc_hbm.at[pl.ds(e0, CHC)], srcb)
                    pltpu.sync_copy(pdst_hbm.at[pl.ds(e0, CHC)], dstb)
                    for g in range(CHC // L):
                        sv = srcb[pl.ds(g * L, L)]
                        dv = dstb[pl.ds(g * L, L)]
                        sv = jnp.clip(sv, 0, N - 1)
                        dv = jnp.clip(dv, 0, N - 1)
                        srcb[pl.ds(g * L, L)] = sv
                        dstb[pl.ds(g * L, L)] = dv
                        pos = e0 + g * L + iota
                        valid = (pos >= lo) & (pos < hi)
                        lv = jnp.clip(dv - b * BUCKET_N, 0, BUCKET_N - 1)
                        locb[pl.ds(g * L, L)] = jnp.where(valid, lv,
                                                          BUCKET_N + s)
                    cp1 = pltpu.async_copy(h_hbm.at[srcb], hb, sem)
                    cp2 = pltpu.async_copy(as_hbm.at[srcb], asb, sem2)
                    cp3 = pltpu.async_copy(ad_hbm.at[dstb], adb, sem3)
                    cp1.wait()
                    cp2.wait()
                    cp3.wait()

                    def edge(r, _):
                        lvec = asb[r, :] + adb[r, :]
                        lvec = jnp.maximum(lvec, 0.2 * lvec)
                        w16 = jnp.exp(lvec - cb16)
                        if heads == 1:
                            w0 = w16[0]
                            for k in range(nchunk):
                                hv = hb[r, pl.ds(k * L, L)]
                                updb[r, pl.ds(k * L, L)] = hv * w0
                            updb[r, pl.ds(HW, L)] = jnp.where(
                                iota == 0, w16, 0.0)
                        else:
                            ws = [w16[h] for h in range(10)] + [jnp.float32(0.0)]
                            for k in range(nchunk):
                                ha = (k * L) // 35
                                bnd = (ha + 1) * 35 - k * L
                                hv = hb[r, pl.ds(k * L, L)]
                                if bnd >= L:
                                    wk = ws[ha]
                                else:
                                    wk = jnp.where(iota < bnd, ws[ha],
                                                   ws[ha + 1])
                                updb[r, pl.ds(k * L, L)] = hv * wk
                            updb[r, pl.ds(HW, L)] = w16
                        return _

                    lax.fori_loop(0, CHC, edge, 0)
                    pltpu.async_copy(updb, acc.at[locb], sem4,
                                     add=True).wait()
                    return _

                lax.fori_loop(0, myn, chunk, 0)
                plsc.subcore_barrier()
                pltpu.sync_copy(
                    acc.at[pl.ds(s * srows, srows)],
                    out_hbm.at[pl.ds(b * BUCKET_N + s * srows, srows)])
                plsc.subcore_barrier()
            return _

        lax.fori_loop(0, NBUCK, bucket, 0)

    return accum


_accum1 = _make_accum(HW1, WP1, 10)
_accum2 = _make_accum(HW2, WP2, 1)


@functools.partial(
    pl.kernel,
    out_type=jax.ShapeDtypeStruct((NTILES, G, 128), jnp.float32),
    mesh=mesh,
    scratch_types=[
        pltpu.VMEM((16, 128), jnp.float32),
        pltpu.VMEM((16,), jnp.int32),
        pltpu.VMEM((G, 128), jnp.float32),
    ],
    compiler_params=_scp,
)
def _pool(hl2_hbm, batch_hbm, out_hbm, hb, bb, acc):
    c = lax.axis_index("c")
    s = lax.axis_index("s")
    w = c * 16 + s
    neg = jnp.full((L,), -3.0e38, jnp.float32)

    def initrow(j, _):
        for q in range(8):
            acc[j, pl.ds(q * L, L)] = neg
        return _

    lax.fori_loop(0, G, initrow, 0)
    r0 = lax.shift_right_logical(w * 3125, 4) * 16
    r1 = jnp.minimum(lax.shift_right_logical((w + 1) * 3125 + 15, 4) * 16,
                     N)
    nch = lax.shift_right_logical(r1 - r0, 4)

    def chunk(i, _):
        rr = r0 + i * 16
        pltpu.sync_copy(hl2_hbm.at[pl.ds(rr, 16)], hb)
        pltpu.sync_copy(batch_hbm.at[pl.ds(rr, 16)], bb)
        bv = jnp.clip(bb[...], 0, G - 1)
        for j in range(16):
            bid = bv[j]
            for q in range(8):
                cur = acc[bid, pl.ds(q * L, L)]
                acc[bid, pl.ds(q * L, L)] = jnp.maximum(
                    cur, hb[j, pl.ds(q * L, L)])
        return _

    lax.fori_loop(0, nch, chunk, 0)
    pltpu.sync_copy(acc, out_hbm.at[w])


def _block_diag_att(att, heads, d, width):
    a = jnp.zeros((16, width), jnp.float32)
    for h in range(heads):
        a = a.at[h, h * d:(h + 1) * d].set(att[h])
    return a.T


def kernel(x, edge_index, batch, W1, att_src1, att_dst1, b1, W2, att_src2,
           att_dst2, b2, fc_W, fc_b, out_W, out_b):
    f32 = jnp.float32
    src = edge_index[0].astype(jnp.int32)
    dst = edge_index[1].astype(jnp.int32)
    loop = jnp.arange(N, dtype=jnp.int32)
    srcp = jnp.concatenate([src, loop, jnp.zeros(E_ALLOC - E, jnp.int32)])
    dstp = jnp.concatenate([dst, loop,
                            jnp.full(E_ALLOC - E, PADV, jnp.int32)])

    dst3 = dstp[:E_PAD].reshape(NTILES, GPT, 16)
    cnt3 = pl.pallas_call(
        _hist_body,
        grid=(NTILES,),
        in_specs=[pl.BlockSpec((1, GPT, 16), lambda t: (t, 0, 0))],
        out_specs=pl.BlockSpec((1, NBINS, 16), lambda t: (t, 0, 0)),
        out_shape=jax.ShapeDtypeStruct((NTILES, NBINS, 16), jnp.int32),
    )(dst3)
    cnt2 = cnt3.transpose(0, 2, 1).reshape(512, NBINS).astype(f32)
    offs, bstart8 = pl.pallas_call(
        _offsets_body,
        out_shape=[jax.ShapeDtypeStruct((512, NBINS), jnp.int32),
                   jax.ShapeDtypeStruct((8, NBINS), jnp.int32)],
    )(cnt2)
    offsT = offs.reshape(32, 16, NBINS).transpose(0, 2, 1).copy()
    bstart = bstart8[0]

    psrc, pdst = _permute(srcp, dstp, offsT)

    W1p = jnp.pad(W1, ((0, 0), (0, 2)))
    As1 = _block_diag_att(att_src1, 10, 35, 352)
    Ad1 = _block_diag_att(att_dst1, 10, 35, 352)
    bm = 1000
    nblk = N // bm
    h1, a1s, a1d, ms3, md3 = pl.pallas_call(
        _mm1_body,
        grid=(nblk,),
        in_specs=[pl.BlockSpec((bm, 35), lambda m: (m, 0)),
                  pl.BlockSpec((35, 352), lambda m: (0, 0)),
                  pl.BlockSpec((352, 16), lambda m: (0, 0)),
                  pl.BlockSpec((352, 16), lambda m: (0, 0))],
        out_specs=[pl.BlockSpec((bm, 352), lambda m: (m, 0)),
                   pl.BlockSpec((bm, 16), lambda m: (m, 0)),
                   pl.BlockSpec((bm, 16), lambda m: (m, 0)),
                   pl.BlockSpec((1, 8, 16), lambda m: (m, 0, 0)),
                   pl.BlockSpec((1, 8, 16), lambda m: (m, 0, 0))],
        out_shape=[jax.ShapeDtypeStruct((N, 352), f32),
                   jax.ShapeDtypeStruct((N, 16), f32),
                   jax.ShapeDtypeStruct((N, 16), f32),
                   jax.ShapeDtypeStruct((nblk, 8, 16), f32),
                   jax.ShapeDtypeStruct((nblk, 8, 16), f32)],
    )(x, W1p, As1, Ad1)
    c1 = pl.pallas_call(
        _cmax_body,
        out_shape=jax.ShapeDtypeStruct((8, 16), f32),
    )(ms3, md3)

    acc1 = _accum1(psrc, pdst, h1, a1s, a1d, bstart, c1)

    rep1 = jnp.zeros((16, 352), f32)
    for h in range(10):
        rep1 = rep1.at[h, h * 35:(h + 1) * 35].set(1.0)
    b1p = jnp.tile(jnp.pad(b1, (0, 2))[None, :], (8, 1))
    W2p = jnp.pad(W2, ((0, 2), (0, 0)))
    A2s = jnp.zeros((128, 16), f32).at[:, 0].set(att_src2[0])
    A2d = jnp.zeros((128, 16), f32).at[:, 0].set(att_dst2[0])
    h2, a2s, a2d, ms23, md23 = pl.pallas_call(
        _norm1mm2_body,
        grid=(nblk,),
        in_specs=[pl.BlockSpec((bm, WP1), lambda m: (m, 0)),
                  pl.BlockSpec((16, 352), lambda m: (0, 0)),
                  pl.BlockSpec((8, 352), lambda m: (0, 0)),
                  pl.BlockSpec((352, 128), lambda m: (0, 0)),
                  pl.BlockSpec((128, 16), lambda m: (0, 0)),
                  pl.BlockSpec((128, 16), lambda m: (0, 0))],
        out_specs=[pl.BlockSpec((bm, 128), lambda m: (m, 0)),
                   pl.BlockSpec((bm, 16), lambda m: (m, 0)),
                   pl.BlockSpec((bm, 16), lambda m: (m, 0)),
                   pl.BlockSpec((1, 8, 16), lambda m: (m, 0, 0)),
                   pl.BlockSpec((1, 8, 16), lambda m: (m, 0, 0))],
        out_shape=[jax.ShapeDtypeStruct((N, 128), f32),
                   jax.ShapeDtypeStruct((N, 16), f32),
                   jax.ShapeDtypeStruct((N, 16), f32),
                   jax.ShapeDtypeStruct((nblk, 8, 16), f32),
                   jax.ShapeDtypeStruct((nblk, 8, 16), f32)],
    )(acc1, rep1, b1p, W2p, A2s, A2d)
    c2 = pl.pallas_call(
        _cmax_body,
        out_shape=jax.ShapeDtypeStruct((8, 16), f32),
    )(ms23, md23)

    acc2 = _accum2(psrc, pdst, h2, a2s, a2d, bstart, c2)

    rep2 = jnp.zeros((16, 128), f32).at[0, :].set(1.0)
    b2t = jnp.tile(b2[None, :], (8, 1))
    hl2 = pl.pallas_call(
        _norm2_body,
        grid=(nblk,),
        in_specs=[pl.BlockSpec((bm, WP2), lambda m: (m, 0)),
                  pl.BlockSpec((16, 128), lambda m: (0, 0)),
                  pl.BlockSpec((8, 128), lambda m: (0, 0))],
        out_specs=pl.BlockSpec((bm, 128), lambda m: (m, 0)),
        out_shape=jax.ShapeDtypeStruct((N, 128), f32),
    )(acc2, rep2, b2t)

    pool2 = _pool(hl2, batch.astype(jnp.int32))
    fcb8 = jnp.tile(fc_b[None, :], (8, 1))
    outWp = jnp.pad(out_W, ((0, 0), (0, 7)))
    outb8 = jnp.tile(jnp.pad(out_b, (0, 7))[None, :], (8, 1))
    out8 = pl.pallas_call(
        _head_body,
        out_shape=jax.ShapeDtypeStruct((G, 8), f32),
    )(pool2, fc_W, fcb8, outWp, outb8)
    return out8[:, :1]
